# Initial kernel scaffold; baseline (speedup 1.0000x reference)
import jax
import jax.numpy as jnp
from jax.experimental import pallas as pl


def kernel(x, weight):
    raise NotImplementedError("write your pallas kernel here")


# speedup vs baseline: 4.3761x; 4.3761x over previous
import numpy as np
import jax
import jax.numpy as jnp
from jax.experimental import pallas as pl

_L = 6
_C = 128
_NO = (_L + 1) ** 2


def _so2_pair_table():
    ret = []
    widx = 0
    for lo in range(_L + 1):
        for li in range(_L + 1):
            mmax = min(lo, li)
            for mw in range(-mmax, mmax + 1):
                if mw != 0:
                    prs = ((-abs(mw), -mw), (abs(mw), mw))
                else:
                    prs = ((0, 0),)
                for mo, mi in prs:
                    ret.append((lo * lo + mo + lo, li * li + mi + li,
                                -1.0 if (mo > 0 and mi < 0) else 1.0, widx))
                widx += 1
    ret.sort()
    return ret, widx


_PAIRS, _NW = _so2_pair_table()
_BY_OUT = {}
for _mo, _mi, _s, _w in _PAIRS:
    _BY_OUT.setdefault(_mo, []).append((_mi, _s, _w))


def _so2_body(x_ref, w_ref, o_ref):
    dn = (((1,), (0,)), ((), ()))
    for mo in range(_NO):
        acc = None
        for mi, s, wi in _BY_OUT[mo]:
            xb = x_ref[:, mi * _C:(mi + 1) * _C]
            wb = w_ref[wi]
            d = jax.lax.dot_general(xb, wb, dn,
                                    preferred_element_type=jnp.float32)
            if acc is None:
                acc = d if s > 0 else -d
            else:
                acc = acc + d if s > 0 else acc - d
        o_ref[:, mo * _C:(mo + 1) * _C] = acc


def kernel(x, weight):
    n = x.shape[0]
    tn = 256
    xf = x.reshape(n, _NO * _C).astype(jnp.bfloat16)
    wf = weight.reshape(_NW, _C, _C).astype(jnp.bfloat16)
    out = pl.pallas_call(
        _so2_body,
        grid=(n // tn,),
        in_specs=[
            pl.BlockSpec((tn, _NO * _C), lambda i: (i, 0)),
            pl.BlockSpec((_NW, _C, _C), lambda i: (0, 0, 0)),
        ],
        out_specs=pl.BlockSpec((tn, _NO * _C), lambda i: (i, 0)),
        out_shape=jax.ShapeDtypeStruct((n, _NO * _C), jnp.float32),
    )(xf, wf)
    return out.reshape(n, _NO, _C)
